# baseline (speedup 1.0000x reference)
import functools
import math

import jax
import jax.numpy as jnp
from jax import lax
from jax.experimental import pallas as pl
from jax.experimental.pallas import tpu as pltpu
from jax.experimental.pallas import tpu_sc as plsc

H = 64
W = 1800
FOV_UP = 3.0 / 180.0 * math.pi
FOV_DOWN = -25.0 / 180.0 * math.pi
FOV = abs(FOV_DOWN) + abs(FOV_UP)
FD = abs(FOV_DOWN)

N = H * W
NT = 16
CPT = 57
S = CPT * 128
M = NT * S
PCT = 64
ROWS = NT * PCT
SENT = N
SLOP = N + 512
FMAX = 3.4028234663852886e38
IR = NT * CPT
HC = 16
NP = 4


def _prep_body(rt_ref, x_ref, y_ref, z_ref,
               pix_ref, dep_ref, tx_ref, ty_ref, tz_ref):
    x = x_ref[...]
    y = y_ref[...]
    z = z_ref[...]
    r00 = rt_ref[0]
    r01 = rt_ref[1]
    r02 = rt_ref[2]
    r10 = rt_ref[3]
    r11 = rt_ref[4]
    r12 = rt_ref[5]
    r20 = rt_ref[6]
    r21 = rt_ref[7]
    r22 = rt_ref[8]
    t0 = rt_ref[9]
    t1 = rt_ref[10]
    t2 = rt_ref[11]

    tx = x * r00 + y * r01 + z * r02 + t0
    ty = x * r10 + y * r11 + z * r12 + t1
    tz = x * r20 + y * r21 + z * r22 + t2

    depth = jnp.sqrt(tx * tx + ty * ty + tz * tz)
    yaw = -lax.atan2(ty, tx)
    ratio = tz / depth
    pitch = 2.0 * lax.atan2(ratio, 1.0 + jnp.sqrt(1.0 - ratio * ratio))

    px = jnp.clip(jnp.floor(0.5 * (yaw / math.pi + 1.0) * W), 0.0, W - 1.0)
    py = jnp.clip(jnp.floor((1.0 - (pitch + FD) / FOV) * H), 0.0, H - 1.0)
    pxi = px.astype(jnp.int32)
    pyi = py.astype(jnp.int32)

    valid = (x != 0.0) & (y != 0.0) & (z != 0.0)
    flat = (lax.broadcasted_iota(jnp.int32, (ROWS, 128), 0) * 128
            + lax.broadcasted_iota(jnp.int32, (ROWS, 128), 1))
    pad = flat >= N

    pix = jnp.where(pad, SLOP, jnp.where(valid, pyi * W + pxi, SENT))
    dep = jnp.where(pad, FMAX, depth)

    pix_ref[...] = pix
    dep_ref[...] = dep
    tx_ref[...] = tx
    ty_ref[...] = ty
    tz_ref[...] = tz


_prep = pl.pallas_call(
    _prep_body,
    out_shape=(
        jax.ShapeDtypeStruct((ROWS, 128), jnp.int32),
        jax.ShapeDtypeStruct((ROWS, 128), jnp.float32),
        jax.ShapeDtypeStruct((ROWS, 128), jnp.float32),
        jax.ShapeDtypeStruct((ROWS, 128), jnp.float32),
        jax.ShapeDtypeStruct((ROWS, 128), jnp.float32),
    ),
    in_specs=[
        pl.BlockSpec(memory_space=pltpu.SMEM),
        pl.BlockSpec(memory_space=pltpu.VMEM),
        pl.BlockSpec(memory_space=pltpu.VMEM),
        pl.BlockSpec(memory_space=pltpu.VMEM),
    ],
)


def _scatter_body(pix_hbm, dep_hbm, x_hbm, y_hbm, z_hbm,
                  rng_out, x_out, y_out, z_out, parts,
                  img, dep_l, pix_l, sem):
    cid = lax.axis_index("c")
    sid = lax.axis_index("s")

    @pl.when(cid == 0)
    def _work():
        fmaxv = jnp.full((16,), FMAX, jnp.float32)

        def init_b(j, carry):
            for v in range(8):
                img[j, pl.ds(v * 16, 16)] = fmaxv
            return carry
        lax.fori_loop(0, IR, init_b, 0)

        for h in range(NP):
            pltpu.sync_copy(dep_hbm.at[sid, pl.ds(h * HC, HC)], dep_l)
            pltpu.sync_copy(pix_hbm.at[sid, pl.ds(h * HC, HC)], pix_l)

            def pt_b(j, carry):
                for v in range(8):
                    d = dep_l[j, pl.ds(v * 16, 16)]
                    p = pix_l[j, pl.ds(v * 16, 16)]
                    r0 = lax.shift_right_logical(p, 7)
                    l0 = lax.bitwise_and(p, 127)

                    def cond(m):
                        return jnp.any(m)

                    def bdy(m):
                        cur = plsc.load_gather(img, [r0, l0], mask=m)
                        want = jnp.logical_and(m, d < cur)
                        plsc.store_scatter(img, [r0, l0], d, mask=want)
                        cur2 = plsc.load_gather(img, [r0, l0], mask=want)
                        return jnp.logical_and(want, d < cur2)

                    lax.while_loop(cond, bdy, jnp.full((16,), True))
                return carry
            lax.fori_loop(0, HC, pt_b, 0)

        pushes = []
        for k in range(NT):
            pushes.append(pltpu.async_copy(
                img.at[pl.ds(k * CPT, CPT)], parts.at[k, sid], sem))
        for d_ in pushes:
            d_.wait()
        plsc.subcore_barrier()

        pulls = []
        for t in range(NT):
            pulls.append(pltpu.async_copy(
                parts.at[sid, t], img.at[pl.ds(t * CPT, CPT)], sem))
        for d_ in pulls:
            d_.wait()

        def mg_b(j, carry):
            for v in range(8):
                acc = img[j, pl.ds(v * 16, 16)]
                for t in range(1, NT):
                    acc = jnp.minimum(acc, img[t * CPT + j, pl.ds(v * 16, 16)])
                accz = jnp.where(acc == FMAX, 0.0, acc)
                img[j, pl.ds(v * 16, 16)] = accz
            return carry
        lax.fori_loop(0, CPT, mg_b, 0)

        outs = []
        for j in range(CPT):
            outs.append(pltpu.async_copy(
                img.at[j], rng_out.at[pl.ds(sid * S + j * 128, 128)], sem))
        for d_ in outs:
            d_.wait()

        zv = jnp.zeros((16,), jnp.float32)

        def z_b(j, carry):
            for v in range(8):
                img[CPT + j, pl.ds(v * 16, 16)] = zv
            return carry
        lax.fori_loop(0, CPT, z_b, 0)

        outs = []
        for j in range(CPT):
            for oref in (x_out, y_out, z_out):
                outs.append(pltpu.async_copy(
                    img.at[CPT + j],
                    oref.at[pl.ds(sid * S + j * 128, 128)], sem))
        for d_ in outs:
            d_.wait()
        plsc.subcore_barrier()

        RV = 2 * CPT
        ST = 2 * CPT + HC
        for h in range(NP):
            pltpu.sync_copy(dep_hbm.at[sid, pl.ds(h * HC, HC)], dep_l)
            pltpu.sync_copy(pix_hbm.at[sid, pl.ds(h * HC, HC)], pix_l)

            gs = []
            for j in range(HC):
                gs.append(pltpu.async_copy(
                    rng_out.at[pix_l.at[j]], img.at[RV + j], sem))
            for d_ in gs:
                d_.wait()

            def w_b(j, carry):
                for v in range(8):
                    d = dep_l[j, pl.ds(v * 16, 16)]
                    rv = img[RV + j, pl.ds(v * 16, 16)]
                    p = pix_l[j, pl.ds(v * 16, 16)]
                    win = d == rv
                    pix_l[j, pl.ds(v * 16, 16)] = jnp.where(
                        win, p, jnp.int32(SLOP))
                return carry
            lax.fori_loop(0, HC, w_b, 0)

            for ch_hbm, ch_out in ((x_hbm, x_out), (y_hbm, y_out),
                                   (z_hbm, z_out)):
                pltpu.sync_copy(ch_hbm.at[sid, pl.ds(h * HC, HC)],
                                img.at[pl.ds(ST, HC)])
                ss = []
                for j in range(HC):
                    ss.append(pltpu.async_copy(
                        img.at[ST + j], ch_out.at[pix_l.at[j]], sem))
                for d_ in ss:
                    d_.wait()


@functools.cache
def _make_scatter():
    return functools.partial(
        pl.kernel,
        out_type=(
            jax.ShapeDtypeStruct((M,), jnp.float32),
            jax.ShapeDtypeStruct((M,), jnp.float32),
            jax.ShapeDtypeStruct((M,), jnp.float32),
            jax.ShapeDtypeStruct((M,), jnp.float32),
            jax.ShapeDtypeStruct((NT, NT, CPT, 128), jnp.float32),
        ),
        mesh=plsc.VectorSubcoreMesh(core_axis_name="c", subcore_axis_name="s"),
        compiler_params=pltpu.CompilerParams(needs_layout_passes=False),
        scratch_types=[
            pltpu.VMEM((IR, 128), jnp.float32),
            pltpu.VMEM((HC, 128), jnp.float32),
            pltpu.VMEM((HC, 128), jnp.int32),
            pltpu.SemaphoreType.DMA,
        ],
    )(_scatter_body)


def _scatter(*args):
    return _make_scatter()(*args)


def _normals(c0, c1, c2, c3):
    chans = (c0, c1, c2, c3)
    top = [ch[0:H - 2, 1:W - 1] - ch[1:H - 1, 1:W - 1] for ch in chans]
    bot = [ch[2:H, 1:W - 1] - ch[1:H - 1, 1:W - 1] for ch in chans]
    lef = [ch[1:H - 1, 0:W - 2] - ch[1:H - 1, 1:W - 1] for ch in chans]
    rig = [ch[1:H - 1, 2:W] - ch[1:H - 1, 1:W - 1] for ch in chans]
    wt = jnp.exp(-0.2 * jnp.abs(top[3]))
    wl = jnp.exp(-0.2 * jnp.abs(lef[3]))
    wb = jnp.exp(-0.2 * jnp.abs(bot[3]))
    wr = jnp.exp(-0.2 * jnp.abs(rig[3]))
    T = [wt * top[i] for i in range(3)]
    L = [wl * lef[i] for i in range(3)]
    B = [wb * bot[i] for i in range(3)]
    R = [wr * rig[i] for i in range(3)]

    def cross(a, b, i):
        j, k = (i + 1) % 3, (i + 2) % 3
        return a[j] * b[k] - a[k] * b[j]

    return [cross(T, L, i) + cross(L, B, i) + cross(B, R, i) + cross(R, T, i)
            for i in range(3)]


def _loss_body(a0_ref, a1_ref, a2_ref, a3_ref,
               r_ref, x_ref, y_ref, z_ref, o_ref):
    n0 = _normals(a0_ref[...], a1_ref[...], a2_ref[...], a3_ref[...])
    r = r_ref[...]
    n1 = _normals(x_ref[...], y_ref[...], z_ref[...], r)
    s1 = jnp.sum(jnp.abs(n1[0] - n0[0])
                 + jnp.abs(n1[1] - n0[1])
                 + jnp.abs(n1[2] - n0[2]))
    da = r[1:H - 1, 1:W - 1] - r[1:H - 1, 2:W]
    db = r[H - 1:H, 1:W - 1] - r[2:H, 1:W - 1]
    s2 = jnp.sum(jnp.exp(jnp.abs(da) + jnp.abs(db)))
    o_ref[0, 0] = s1 * s2


_loss = pl.pallas_call(
    _loss_body,
    out_shape=jax.ShapeDtypeStruct((1, 1), jnp.float32),
    out_specs=pl.BlockSpec(memory_space=pltpu.SMEM),
)


def kernel(pred_x, pred_q, pred_mask0, pred_mask1, imgs_0, imgs_1,
           gt_x, gt_q):
    q = pred_q[0]
    q = q / jnp.linalg.norm(q)
    w, x, y, z = q[0], q[1], q[2], q[3]
    row0 = jnp.stack([1.0 - 2.0 * (y * y + z * z), 2.0 * (x * y - w * z),
                      2.0 * (x * z + w * y)])
    row1 = jnp.stack([2.0 * (x * y + w * z), 1.0 - 2.0 * (x * x + z * z),
                      2.0 * (y * z - w * x)])
    row2 = jnp.stack([2.0 * (x * z - w * y), 2.0 * (y * z + w * x),
                      1.0 - 2.0 * (x * x + y * y)])
    rt = jnp.concatenate([row0, row1, row2, pred_x[0],
                          jnp.zeros((4,), jnp.float32)]).astype(jnp.float32)

    def plane(c):
        p = imgs_1[0, c].reshape(-1)
        return jnp.pad(p, (0, ROWS * 128 - N)).reshape(ROWS, 128)

    pix, dep, tx, ty, tz = _prep(rt, plane(0), plane(1), plane(2))

    def t3(a):
        return a.reshape(NT, PCT, 128)

    rng_f, xf, yf, zf, _ = _scatter(t3(pix), t3(dep), t3(tx), t3(ty), t3(tz))

    r_img = rng_f[:N].reshape(H, W)
    x_img = xf[:N].reshape(H, W)
    y_img = yf[:N].reshape(H, W)
    z_img = zf[:N].reshape(H, W)

    out = _loss(imgs_0[0, 0], imgs_0[0, 1], imgs_0[0, 2], imgs_0[0, 3],
                r_img, x_img, y_img, z_img)
    return out[0, 0]
